# baseline (speedup 1.0000x reference)
import functools

import jax
import jax.numpy as jnp
from jax import lax
from jax.experimental import pallas as pl
from jax.experimental.pallas import tpu as pltpu
from jax.experimental.pallas import tpu_sc as plsc

N = 10000
E = 320000
H1 = 64
H2 = 128
NCLS = 40
H3P = 48

NC = 2
NS = 16
NW = NC * NS
EPW = 10240
E_PAD = NW * EPW
CH = 128
NCHUNK = EPW // CH
NBUF = 4
N_PAD = 10240
RPS = N_PAD // NS


def _sc_mesh():
    return plsc.VectorSubcoreMesh(core_axis_name="c", subcore_axis_name="s")


_SC_PARAMS = pltpu.CompilerParams(use_tc_tiling_on_sc=False)


def _sc_degree(dst_r, zeros16, ones16):
    @functools.partial(
        pl.kernel,
        out_type=jax.ShapeDtypeStruct((NC, N_PAD, 16), jnp.float32),
        mesh=_sc_mesh(),
        compiler_params=_SC_PARAMS,
        scratch_types=[
            pltpu.VMEM((NCHUNK, CH), jnp.int32),
            pltpu.VMEM((CH, 16), jnp.float32),
            pltpu.VMEM_SHARED((N_PAD, 16), jnp.float32),
            pltpu.SemaphoreType.DMA,
        ],
    )
    def deg_kernel(dst_hbm, z_hbm, one_hbm, out_hbm, didx_v, ones_v, cnt_sh, dsem):
        c = lax.axis_index("c")
        s = lax.axis_index("s")
        wid = s * NC + c
        row0 = s * RPS
        pltpu.sync_copy(dst_hbm.at[wid], didx_v)
        pltpu.sync_copy(one_hbm, ones_v)
        pltpu.sync_copy(z_hbm.at[pl.ds(row0, RPS)], cnt_sh.at[pl.ds(row0, RPS)])
        plsc.subcore_barrier()

        @pl.loop(0, NCHUNK)
        def _(j):
            pltpu.async_copy(ones_v, cnt_sh.at[didx_v.at[j]], dsem, add=True)

        @pl.loop(0, NCHUNK)
        def _(j):
            pltpu.make_async_copy(ones_v, cnt_sh.at[didx_v.at[j]], dsem).wait()

        plsc.subcore_barrier()
        pltpu.sync_copy(cnt_sh.at[pl.ds(row0, RPS)], out_hbm.at[c, pl.ds(row0, RPS)])

    return deg_kernel(dst_r, zeros16, ones16)


def _sc_aggregate(u_pad, src_r, dst_r, zeros, nbuf):
    H = u_pad.shape[1]
    nchunk, ch = src_r.shape[1], src_r.shape[2]

    @functools.partial(
        pl.kernel,
        out_type=jax.ShapeDtypeStruct((NC, N_PAD, H), jnp.float32),
        mesh=_sc_mesh(),
        compiler_params=_SC_PARAMS,
        scratch_types=[
            pltpu.VMEM((nchunk, ch), jnp.int32),
            pltpu.VMEM((nchunk, ch), jnp.int32),
            pltpu.VMEM((nbuf, ch, H), jnp.float32),
            pltpu.VMEM_SHARED((N_PAD, H), jnp.float32),
        ]
        + [pltpu.SemaphoreType.DMA] * (2 * nbuf),
    )
    def agg_kernel(u_hbm, s_hbm, d_hbm, z_hbm, out_hbm, sidx_v, didx_v, rows_v,
                   agg_sh, *sems):
        c = lax.axis_index("c")
        s = lax.axis_index("s")
        wid = s * NC + c
        row0 = s * RPS
        pltpu.sync_copy(s_hbm.at[wid], sidx_v)
        pltpu.sync_copy(d_hbm.at[wid], didx_v)
        pltpu.sync_copy(z_hbm.at[pl.ds(row0, RPS)], agg_sh.at[pl.ds(row0, RPS)])
        plsc.subcore_barrier()

        gsem, ssem = sems[:nbuf], sems[nbuf:]

        def gather_start(b, jj):
            pltpu.async_copy(u_hbm.at[sidx_v.at[jj]], rows_v.at[b], gsem[b])

        def gather_wait(b, jj):
            pltpu.make_async_copy(u_hbm.at[sidx_v.at[jj]], rows_v.at[b],
                                  gsem[b]).wait()

        def scatter_start(b, jj):
            pltpu.async_copy(rows_v.at[b], agg_sh.at[didx_v.at[jj]], ssem[b],
                             add=True)

        def scatter_wait(b, jj):
            pltpu.make_async_copy(rows_v.at[b], agg_sh.at[didx_v.at[jj]],
                                  ssem[b]).wait()

        for b in range(nbuf):
            gather_start(b, b)

        @pl.loop(0, nchunk, step=nbuf)
        def _(j):
            for b in range(nbuf):
                jj = j + b
                gather_wait(b, jj)
                scatter_start(b, jj)
                pb = (b - 1) % nbuf
                pjj = jj - 1

                @pl.when(pjj >= 0)
                def _():
                    scatter_wait(pb, pjj)

                    @pl.when(pjj + nbuf < nchunk)
                    def _():
                        gather_start(pb, pjj + nbuf)

        scatter_wait(nbuf - 1, nchunk - 1)
        plsc.subcore_barrier()
        pltpu.sync_copy(agg_sh.at[pl.ds(row0, RPS)], out_hbm.at[c, pl.ds(row0, RPS)])

    return agg_kernel(u_pad, src_r, dst_r, zeros)


def _tc_matmul(x, W):
    def body(x_ref, w_ref, o_ref):
        o_ref[...] = jnp.dot(x_ref[...], w_ref[...], preferred_element_type=jnp.float32)

    return pl.pallas_call(
        body,
        out_shape=jax.ShapeDtypeStruct((x.shape[0], W.shape[1]), jnp.float32),
    )(x, W)


def _tc_dinv_scale(deg_parts, h):
    def body(deg_ref, h_ref, dinv_ref, u_ref):
        deg = deg_ref[0, :N, 0:1] + deg_ref[1, :N, 0:1] + 1.0
        dinv = lax.rsqrt(deg)
        dinv_ref[...] = dinv
        u_ref[...] = h_ref[...] * dinv

    return pl.pallas_call(
        body,
        out_shape=(
            jax.ShapeDtypeStruct((N, 1), jnp.float32),
            jax.ShapeDtypeStruct(h.shape, jnp.float32),
        ),
    )(deg_parts, h)


def _tc_post(agg, u, dinv, b, g, bt, Wn):
    def body(agg_ref, u_ref, dinv_ref, b_ref, g_ref, bt_ref, w_ref, o_ref):
        dinv = dinv_ref[...]
        z = (agg_ref[0, :N] + agg_ref[1, :N] + u_ref[...]) * dinv + b_ref[...]
        m = jnp.mean(z, axis=0, keepdims=True)
        zc = z - m
        v = jnp.mean(zc * zc, axis=0, keepdims=True)
        hh = jnp.maximum(zc * lax.rsqrt(v + 1e-5) * g_ref[...] + bt_ref[...], 0.0)
        o_ref[...] = jnp.dot(hh, w_ref[...], preferred_element_type=jnp.float32) * dinv

    return pl.pallas_call(
        body,
        out_shape=jax.ShapeDtypeStruct((N, Wn.shape[1]), jnp.float32),
    )(agg, u, dinv, b, g, bt, Wn)


def _tc_out(agg, u, dinv, b):
    def body(agg_ref, u_ref, dinv_ref, b_ref, o_ref):
        z = (agg_ref[0, :N, :NCLS] + agg_ref[1, :N, :NCLS] + u_ref[...]) * dinv_ref[...] + b_ref[...]
        mx = jnp.max(z, axis=1, keepdims=True)
        lse = jnp.log(jnp.sum(jnp.exp(z - mx), axis=1, keepdims=True)) + mx
        o_ref[...] = z - lse

    return pl.pallas_call(
        body,
        out_shape=jax.ShapeDtypeStruct((N, NCLS), jnp.float32),
    )(agg, u, dinv, b)


def kernel(x, edge_index, W1, b1, g1, bt1, W2, b2, g2, bt2, W3, b3):
    src = edge_index[0]
    dst = edge_index[1]
    pad = N + (jnp.arange(E_PAD - E, dtype=jnp.int32) % (N_PAD - N))
    src_flat = jnp.concatenate([src, pad])
    dst_flat = jnp.concatenate([dst, pad])
    src_r = src_flat.reshape(NW, NCHUNK, CH)
    dst_r = dst_flat.reshape(NW, NCHUNK, CH)
    src_r64 = src_flat.reshape(NW, 2 * NCHUNK, CH // 2)
    dst_r64 = dst_flat.reshape(NW, 2 * NCHUNK, CH // 2)

    zeros16 = jnp.zeros((N_PAD, 16), jnp.float32)
    ones16 = jnp.ones((CH, 16), jnp.float32)
    z1 = jnp.zeros((N_PAD, H1), jnp.float32)
    z2 = jnp.zeros((N_PAD, H2), jnp.float32)
    z3 = jnp.zeros((N_PAD, H3P), jnp.float32)

    h1 = _tc_matmul(x, W1)
    deg_parts = _sc_degree(dst_r, zeros16, ones16)
    dinv, u1 = _tc_dinv_scale(deg_parts, h1)

    u1p = jnp.pad(u1, ((0, N_PAD - N), (0, 0)))
    agg1 = _sc_aggregate(u1p, src_r, dst_r, z1, nbuf=4)
    u2 = _tc_post(agg1, u1, dinv, b1.reshape(1, -1), g1.reshape(1, -1),
                  bt1.reshape(1, -1), W2)

    u2p = jnp.pad(u2, ((0, N_PAD - N), (0, 0)))
    agg2 = _sc_aggregate(u2p, src_r64, dst_r64, z2, nbuf=2)
    u3 = _tc_post(agg2, u2, dinv, b2.reshape(1, -1), g2.reshape(1, -1),
                  bt2.reshape(1, -1), W3)

    u3p = jnp.pad(u3, ((0, N_PAD - N), (0, H3P - NCLS)))
    agg3 = _sc_aggregate(u3p, src_r, dst_r, z3, nbuf=4)
    return _tc_out(agg3, u3, dinv, b3.reshape(1, -1))


# speedup vs baseline: 1.1493x; 1.1493x over previous
import functools

import jax
import jax.numpy as jnp
from jax import lax
from jax.experimental import pallas as pl
from jax.experimental.pallas import tpu as pltpu
from jax.experimental.pallas import tpu_sc as plsc

N = 10000
E = 320000
H1 = 64
H2 = 128
NCLS = 40
H3P = 48

NC = 2
NS = 16
NW = NC * NS
EPW = 10240
E_PAD = NW * EPW
CH = 128
NCHUNK = EPW // CH
NBUF = 4
N_PAD = 10240
RPS = N_PAD // NS


def _sc_mesh():
    return plsc.VectorSubcoreMesh(core_axis_name="c", subcore_axis_name="s")


_SC_PARAMS = pltpu.CompilerParams(use_tc_tiling_on_sc=False)


def _sc_degree(dst_r, zeros16, ones16):
    @functools.partial(
        pl.kernel,
        out_type=jax.ShapeDtypeStruct((NC, N_PAD, 16), jnp.float32),
        mesh=_sc_mesh(),
        compiler_params=_SC_PARAMS,
        scratch_types=[
            pltpu.VMEM((NCHUNK, CH), jnp.int32),
            pltpu.VMEM((CH, 16), jnp.float32),
            pltpu.VMEM_SHARED((N_PAD, 16), jnp.float32),
            pltpu.SemaphoreType.DMA,
        ],
    )
    def deg_kernel(dst_hbm, z_hbm, one_hbm, out_hbm, didx_v, ones_v, cnt_sh, dsem):
        c = lax.axis_index("c")
        s = lax.axis_index("s")
        wid = s * NC + c
        row0 = s * RPS
        pltpu.sync_copy(dst_hbm.at[wid], didx_v)
        pltpu.sync_copy(one_hbm, ones_v)
        pltpu.sync_copy(z_hbm.at[pl.ds(row0, RPS)], cnt_sh.at[pl.ds(row0, RPS)])
        plsc.subcore_barrier()

        @pl.loop(0, NCHUNK)
        def _(j):
            pltpu.async_copy(ones_v, cnt_sh.at[didx_v.at[j]], dsem, add=True)

        @pl.loop(0, NCHUNK)
        def _(j):
            pltpu.make_async_copy(ones_v, cnt_sh.at[didx_v.at[j]], dsem).wait()

        plsc.subcore_barrier()
        pltpu.sync_copy(cnt_sh.at[pl.ds(row0, RPS)], out_hbm.at[c, pl.ds(row0, RPS)])

    return deg_kernel(dst_r, zeros16, ones16)


def _sc_aggregate(u_pad, src_r, dst_r, zeros, nbuf):
    H = u_pad.shape[1]
    nchunk, ch = src_r.shape[1], src_r.shape[2]

    @functools.partial(
        pl.kernel,
        out_type=jax.ShapeDtypeStruct((NC, N_PAD, H), jnp.float32),
        mesh=_sc_mesh(),
        compiler_params=_SC_PARAMS,
        scratch_types=[
            pltpu.VMEM((nchunk, ch), jnp.int32),
            pltpu.VMEM((nchunk, ch), jnp.int32),
            pltpu.VMEM((nbuf, ch, H), jnp.float32),
            pltpu.VMEM_SHARED((N_PAD, H), jnp.float32),
        ]
        + [pltpu.SemaphoreType.DMA] * (2 * nbuf),
    )
    def agg_kernel(u_hbm, s_hbm, d_hbm, z_hbm, out_hbm, sidx_v, didx_v, rows_v,
                   agg_sh, *sems):
        c = lax.axis_index("c")
        s = lax.axis_index("s")
        wid = s * NC + c
        row0 = s * RPS
        pltpu.sync_copy(s_hbm.at[wid], sidx_v)
        pltpu.sync_copy(d_hbm.at[wid], didx_v)
        pltpu.sync_copy(z_hbm.at[pl.ds(row0, RPS)], agg_sh.at[pl.ds(row0, RPS)])
        plsc.subcore_barrier()

        gsem, ssem = sems[:nbuf], sems[nbuf:]

        def gather_start(b, jj):
            pltpu.async_copy(u_hbm.at[sidx_v.at[jj]], rows_v.at[b], gsem[b])

        def gather_wait(b, jj):
            pltpu.make_async_copy(u_hbm.at[sidx_v.at[jj]], rows_v.at[b],
                                  gsem[b]).wait()

        def scatter_start(b, jj):
            pltpu.async_copy(rows_v.at[b], agg_sh.at[didx_v.at[jj]], ssem[b],
                             add=True)

        def scatter_wait(b, jj):
            pltpu.make_async_copy(rows_v.at[b], agg_sh.at[didx_v.at[jj]],
                                  ssem[b]).wait()

        for b in range(nbuf):
            gather_start(b, b)

        @pl.loop(0, nchunk - nbuf, step=nbuf)
        def _(j):
            for b in range(nbuf):
                jj = j + b
                gather_wait(b, jj)
                scatter_start(b, jj)
                scatter_wait(b, jj)
                gather_start(b, jj + nbuf)

        for b in range(nbuf):
            jj = nchunk - nbuf + b
            gather_wait(b, jj)
            scatter_start(b, jj)
            scatter_wait(b, jj)
        plsc.subcore_barrier()
        pltpu.sync_copy(agg_sh.at[pl.ds(row0, RPS)], out_hbm.at[c, pl.ds(row0, RPS)])

    return agg_kernel(u_pad, src_r, dst_r, zeros)


def _tc_matmul(x, W):
    def body(x_ref, w_ref, o_ref):
        o_ref[...] = jnp.dot(x_ref[...], w_ref[...], preferred_element_type=jnp.float32)

    return pl.pallas_call(
        body,
        out_shape=jax.ShapeDtypeStruct((x.shape[0], W.shape[1]), jnp.float32),
    )(x, W)


def _tc_dinv_scale(deg_parts, h):
    def body(deg_ref, h_ref, dinv_ref, u_ref):
        deg = deg_ref[0, :N, 0:1] + deg_ref[1, :N, 0:1] + 1.0
        dinv = lax.rsqrt(deg)
        dinv_ref[...] = dinv
        u_ref[...] = h_ref[...] * dinv

    return pl.pallas_call(
        body,
        out_shape=(
            jax.ShapeDtypeStruct((N, 1), jnp.float32),
            jax.ShapeDtypeStruct(h.shape, jnp.float32),
        ),
    )(deg_parts, h)


def _tc_post(agg, u, dinv, b, g, bt, Wn):
    def body(agg_ref, u_ref, dinv_ref, b_ref, g_ref, bt_ref, w_ref, o_ref):
        dinv = dinv_ref[...]
        z = (agg_ref[0, :N] + agg_ref[1, :N] + u_ref[...]) * dinv + b_ref[...]
        m = jnp.mean(z, axis=0, keepdims=True)
        zc = z - m
        v = jnp.mean(zc * zc, axis=0, keepdims=True)
        hh = jnp.maximum(zc * lax.rsqrt(v + 1e-5) * g_ref[...] + bt_ref[...], 0.0)
        o_ref[...] = jnp.dot(hh, w_ref[...], preferred_element_type=jnp.float32) * dinv

    return pl.pallas_call(
        body,
        out_shape=jax.ShapeDtypeStruct((N, Wn.shape[1]), jnp.float32),
    )(agg, u, dinv, b, g, bt, Wn)


def _tc_out(agg, u, dinv, b):
    def body(agg_ref, u_ref, dinv_ref, b_ref, o_ref):
        z = (agg_ref[0, :N, :NCLS] + agg_ref[1, :N, :NCLS] + u_ref[...]) * dinv_ref[...] + b_ref[...]
        mx = jnp.max(z, axis=1, keepdims=True)
        lse = jnp.log(jnp.sum(jnp.exp(z - mx), axis=1, keepdims=True)) + mx
        o_ref[...] = z - lse

    return pl.pallas_call(
        body,
        out_shape=jax.ShapeDtypeStruct((N, NCLS), jnp.float32),
    )(agg, u, dinv, b)


def kernel(x, edge_index, W1, b1, g1, bt1, W2, b2, g2, bt2, W3, b3):
    src = edge_index[0]
    dst = edge_index[1]
    pad = N + (jnp.arange(E_PAD - E, dtype=jnp.int32) % (N_PAD - N))
    src_flat = jnp.concatenate([src, pad])
    dst_flat = jnp.concatenate([dst, pad])
    src_r = src_flat.reshape(NW, NCHUNK, CH)
    dst_r = dst_flat.reshape(NW, NCHUNK, CH)
    src_r64 = src_flat.reshape(NW, 2 * NCHUNK, CH // 2)
    dst_r64 = dst_flat.reshape(NW, 2 * NCHUNK, CH // 2)

    zeros16 = jnp.zeros((N_PAD, 16), jnp.float32)
    ones16 = jnp.ones((CH, 16), jnp.float32)
    z1 = jnp.zeros((N_PAD, H1), jnp.float32)
    z2 = jnp.zeros((N_PAD, H2), jnp.float32)
    z3 = jnp.zeros((N_PAD, H3P), jnp.float32)

    h1 = _tc_matmul(x, W1)
    deg_parts = _sc_degree(dst_r, zeros16, ones16)
    dinv, u1 = _tc_dinv_scale(deg_parts, h1)

    u1p = jnp.pad(u1, ((0, N_PAD - N), (0, 0)))
    agg1 = _sc_aggregate(u1p, src_r, dst_r, z1, nbuf=4)
    u2 = _tc_post(agg1, u1, dinv, b1.reshape(1, -1), g1.reshape(1, -1),
                  bt1.reshape(1, -1), W2)

    u2p = jnp.pad(u2, ((0, N_PAD - N), (0, 0)))
    agg2 = _sc_aggregate(u2p, src_r64, dst_r64, z2, nbuf=2)
    u3 = _tc_post(agg2, u2, dinv, b2.reshape(1, -1), g2.reshape(1, -1),
                  bt2.reshape(1, -1), W3)

    u3p = jnp.pad(u3, ((0, N_PAD - N), (0, H3P - NCLS)))
    agg3 = _sc_aggregate(u3p, src_r, dst_r, z3, nbuf=4)
    return _tc_out(agg3, u3, dinv, b3.reshape(1, -1))


# speedup vs baseline: 1.2965x; 1.1281x over previous
import functools

import jax
import jax.numpy as jnp
from jax import lax
from jax.experimental import pallas as pl
from jax.experimental.pallas import tpu as pltpu
from jax.experimental.pallas import tpu_sc as plsc

N = 10000
E = 320000
H1 = 64
H2 = 128
NCLS = 40
H3P = 48

NC = 2
NS = 16
NW = NC * NS
EPW = E // NW
CH = 80
NCHUNK = EPW // CH
N_PAD = 10240
RPS = N_PAD // NS


def _sc_mesh():
    return plsc.VectorSubcoreMesh(core_axis_name="c", subcore_axis_name="s")


_SC_PARAMS = pltpu.CompilerParams(use_tc_tiling_on_sc=False)


def _sc_degree(dst_r, zeros16, ones16):
    @functools.partial(
        pl.kernel,
        out_type=jax.ShapeDtypeStruct((NC, N_PAD, 16), jnp.float32),
        mesh=_sc_mesh(),
        compiler_params=_SC_PARAMS,
        scratch_types=[
            pltpu.VMEM((NCHUNK, CH), jnp.int32),
            pltpu.VMEM((CH, 16), jnp.float32),
            pltpu.VMEM_SHARED((N_PAD, 16), jnp.float32),
            pltpu.SemaphoreType.DMA,
        ],
    )
    def deg_kernel(dst_hbm, z_hbm, one_hbm, out_hbm, didx_v, ones_v, cnt_sh, dsem):
        c = lax.axis_index("c")
        s = lax.axis_index("s")
        wid = s * NC + c
        row0 = s * RPS
        pltpu.sync_copy(dst_hbm.at[wid], didx_v)
        pltpu.sync_copy(one_hbm, ones_v)
        pltpu.sync_copy(z_hbm.at[pl.ds(row0, RPS)], cnt_sh.at[pl.ds(row0, RPS)])
        plsc.subcore_barrier()

        @pl.loop(0, NCHUNK)
        def _(j):
            pltpu.async_copy(ones_v, cnt_sh.at[didx_v.at[j]], dsem, add=True)

        @pl.loop(0, NCHUNK)
        def _(j):
            pltpu.make_async_copy(ones_v, cnt_sh.at[didx_v.at[j]], dsem).wait()

        plsc.subcore_barrier()
        pltpu.sync_copy(cnt_sh.at[pl.ds(row0, RPS)], out_hbm.at[c, pl.ds(row0, RPS)])

    return deg_kernel(dst_r, zeros16, ones16)


def _sc_aggregate(u_pad, src_r, dst_r, zeros, nbuf):
    H = u_pad.shape[1]
    nchunk, ch = src_r.shape[1], src_r.shape[2]

    @functools.partial(
        pl.kernel,
        out_type=jax.ShapeDtypeStruct((NC, N_PAD, H), jnp.float32),
        mesh=_sc_mesh(),
        compiler_params=_SC_PARAMS,
        scratch_types=[
            pltpu.VMEM((nchunk, ch), jnp.int32),
            pltpu.VMEM((nchunk, ch), jnp.int32),
            pltpu.VMEM((nbuf, ch, H), jnp.float32),
            pltpu.VMEM_SHARED((N_PAD, H), jnp.float32),
        ]
        + [pltpu.SemaphoreType.DMA] * (2 * nbuf),
    )
    def agg_kernel(u_hbm, s_hbm, d_hbm, z_hbm, out_hbm, sidx_v, didx_v, rows_v,
                   agg_sh, *sems):
        c = lax.axis_index("c")
        s = lax.axis_index("s")
        wid = s * NC + c
        row0 = s * RPS
        pltpu.sync_copy(s_hbm.at[wid], sidx_v)
        pltpu.sync_copy(d_hbm.at[wid], didx_v)
        pltpu.sync_copy(z_hbm.at[pl.ds(row0, RPS)], agg_sh.at[pl.ds(row0, RPS)])
        plsc.subcore_barrier()

        gsem, ssem = sems[:nbuf], sems[nbuf:]

        def gather_start(b, jj):
            pltpu.async_copy(u_hbm.at[sidx_v.at[jj]], rows_v.at[b], gsem[b])

        def gather_wait(b, jj):
            pltpu.make_async_copy(u_hbm.at[sidx_v.at[jj]], rows_v.at[b],
                                  gsem[b]).wait()

        def scatter_start(b, jj):
            pltpu.async_copy(rows_v.at[b], agg_sh.at[didx_v.at[jj]], ssem[b],
                             add=True)

        def scatter_wait(b, jj):
            pltpu.make_async_copy(rows_v.at[b], agg_sh.at[didx_v.at[jj]],
                                  ssem[b]).wait()

        for b in range(nbuf):
            gather_start(b, b)

        @pl.loop(0, nchunk - nbuf, step=nbuf)
        def _(j):
            for b in range(nbuf):
                jj = j + b
                gather_wait(b, jj)
                scatter_start(b, jj)
                scatter_wait(b, jj)
                gather_start(b, jj + nbuf)

        for b in range(nbuf):
            jj = nchunk - nbuf + b
            gather_wait(b, jj)
            scatter_start(b, jj)
            scatter_wait(b, jj)
        plsc.subcore_barrier()
        pltpu.sync_copy(agg_sh.at[pl.ds(row0, RPS)], out_hbm.at[c, pl.ds(row0, RPS)])

    return agg_kernel(u_pad, src_r, dst_r, zeros)


def _tc_matmul(x, W):
    def body(x_ref, w_ref, o_ref):
        o_ref[...] = jnp.dot(x_ref[...], w_ref[...], preferred_element_type=jnp.float32)

    return pl.pallas_call(
        body,
        out_shape=jax.ShapeDtypeStruct((x.shape[0], W.shape[1]), jnp.float32),
    )(x, W)


def _tc_dinv_scale(deg_parts, h):
    def body(deg_ref, h_ref, dinv_ref, u_ref):
        deg = deg_ref[0, :N, 0:1] + deg_ref[1, :N, 0:1] + 1.0
        dinv = lax.rsqrt(deg)
        dinv_ref[...] = dinv
        u_ref[...] = h_ref[...] * dinv

    return pl.pallas_call(
        body,
        out_shape=(
            jax.ShapeDtypeStruct((N, 1), jnp.float32),
            jax.ShapeDtypeStruct(h.shape, jnp.float32),
        ),
    )(deg_parts, h)


def _tc_post(agg, u, dinv, b, g, bt, Wn):
    def body(agg_ref, u_ref, dinv_ref, b_ref, g_ref, bt_ref, w_ref, o_ref):
        dinv = dinv_ref[...]
        z = (agg_ref[0, :N] + agg_ref[1, :N] + u_ref[...]) * dinv + b_ref[...]
        m = jnp.mean(z, axis=0, keepdims=True)
        zc = z - m
        v = jnp.mean(zc * zc, axis=0, keepdims=True)
        hh = jnp.maximum(zc * lax.rsqrt(v + 1e-5) * g_ref[...] + bt_ref[...], 0.0)
        o_ref[...] = jnp.dot(hh, w_ref[...], preferred_element_type=jnp.float32) * dinv

    return pl.pallas_call(
        body,
        out_shape=jax.ShapeDtypeStruct((N, Wn.shape[1]), jnp.float32),
    )(agg, u, dinv, b, g, bt, Wn)


def _tc_out(agg, u, dinv, b):
    def body(agg_ref, u_ref, dinv_ref, b_ref, o_ref):
        z = (agg_ref[0, :N, :NCLS] + agg_ref[1, :N, :NCLS] + u_ref[...]) * dinv_ref[...] + b_ref[...]
        mx = jnp.max(z, axis=1, keepdims=True)
        lse = jnp.log(jnp.sum(jnp.exp(z - mx), axis=1, keepdims=True)) + mx
        o_ref[...] = z - lse

    return pl.pallas_call(
        body,
        out_shape=jax.ShapeDtypeStruct((N, NCLS), jnp.float32),
    )(agg, u, dinv, b)


def kernel(x, edge_index, W1, b1, g1, bt1, W2, b2, g2, bt2, W3, b3):
    src_r = edge_index[0].reshape(NW, NCHUNK, CH)
    dst_r = edge_index[1].reshape(NW, NCHUNK, CH)
    src_r40 = edge_index[0].reshape(NW, 2 * NCHUNK, CH // 2)
    dst_r40 = edge_index[1].reshape(NW, 2 * NCHUNK, CH // 2)

    zeros16 = jnp.zeros((N_PAD, 16), jnp.float32)
    ones16 = jnp.ones((CH, 16), jnp.float32)
    z1 = jnp.zeros((N_PAD, H1), jnp.float32)
    z2 = jnp.zeros((N_PAD, H2), jnp.float32)
    z3 = jnp.zeros((N_PAD, H3P), jnp.float32)

    h1 = _tc_matmul(x, W1)
    deg_parts = _sc_degree(dst_r, zeros16, ones16)
    dinv, u1 = _tc_dinv_scale(deg_parts, h1)

    agg1 = _sc_aggregate(u1, src_r, dst_r, z1, nbuf=5)
    u2 = _tc_post(agg1, u1, dinv, b1.reshape(1, -1), g1.reshape(1, -1),
                  bt1.reshape(1, -1), W2)

    agg2 = _sc_aggregate(u2, src_r40, dst_r40, z2, nbuf=5)
    u3 = _tc_post(agg2, u2, dinv, b2.reshape(1, -1), g2.reshape(1, -1),
                  bt2.reshape(1, -1), W3)

    u3p = jnp.pad(u3, ((0, 0), (0, H3P - NCLS)))
    agg3 = _sc_aggregate(u3p, src_r, dst_r, z3, nbuf=5)
    return _tc_out(agg3, u3, dinv, b3.reshape(1, -1))
